# Initial kernel scaffold; baseline (speedup 1.0000x reference)
import jax
import jax.numpy as jnp
from jax.experimental import pallas as pl


def kernel(x, edge_index, batch, W1, b1, W2, b2):
    raise NotImplementedError("write your pallas kernel here")


# speedup vs baseline: 6.4412x; 6.4412x over previous
import functools

import jax
import jax.numpy as jnp
from jax import lax
from jax.experimental import pallas as pl
from jax.experimental.pallas import tpu as pltpu

N_NODES = 100000
N_GRAPHS = 512
D_FEAT = 128
HIDDEN = 256
D_TARGET = 64

BLK = 2000
NBLK = N_NODES // BLK


def _seg_mlp_body(batch_ref, x_ref, w1_ref, b1_ref, w2_ref, b2_ref,
                  out_ref, acc_ref):
    i = pl.program_id(0)

    @pl.when(i == 0)
    def _init():
        acc_ref[...] = jnp.zeros_like(acc_ref)

    seg = lax.broadcasted_iota(jnp.int32, (N_GRAPHS, BLK), 0)
    ids = batch_ref[0]
    onehot = (seg == ids).astype(jnp.float32)
    acc_ref[...] += jnp.dot(onehot, x_ref[...],
                            preferred_element_type=jnp.float32)

    @pl.when(i == NBLK - 1)
    def _head():
        pooled = acc_ref[...]
        h = lax.dot_general(pooled, w1_ref[...], (((1,), (1,)), ((), ())),
                            preferred_element_type=jnp.float32)
        h = jnp.maximum(h + b1_ref[...], 0.0)
        o = lax.dot_general(h, w2_ref[...], (((1,), (1,)), ((), ())),
                            preferred_element_type=jnp.float32)
        out_ref[...] = o + b2_ref[...]


def kernel(x, edge_index, batch, W1, b1, W2, b2):
    del edge_index
    batch3d = batch.astype(jnp.int32).reshape(NBLK, 1, BLK)
    grid = (NBLK,)
    out = pl.pallas_call(
        _seg_mlp_body,
        grid=grid,
        in_specs=[
            pl.BlockSpec((1, 1, BLK), lambda i: (i, 0, 0)),
            pl.BlockSpec((BLK, D_FEAT), lambda i: (i, 0)),
            pl.BlockSpec((HIDDEN, D_FEAT), lambda i: (0, 0)),
            pl.BlockSpec((1, HIDDEN), lambda i: (0, 0)),
            pl.BlockSpec((D_TARGET, HIDDEN), lambda i: (0, 0)),
            pl.BlockSpec((1, D_TARGET), lambda i: (0, 0)),
        ],
        out_specs=pl.BlockSpec((N_GRAPHS, D_TARGET), lambda i: (0, 0)),
        out_shape=jax.ShapeDtypeStruct((N_GRAPHS, D_TARGET), jnp.float32),
        scratch_shapes=[pltpu.VMEM((N_GRAPHS, D_FEAT), jnp.float32)],
    )(batch3d, x, W1, b1.reshape(1, HIDDEN), W2, b2.reshape(1, D_TARGET))
    return out
